# baseline (speedup 1.0000x reference)
import functools

import jax
import jax.numpy as jnp
from jax import lax
from jax.experimental import pallas as pl
from jax.experimental.pallas import tpu as pltpu
from jax.experimental.pallas import tpu_sc as plsc

BATCH = 16384
LANES = 16


def _irt_formula(th, a, b, c):
    cs = 1.0 / (1.0 + jnp.exp(-c))
    t = jnp.exp(-jnp.abs(a))
    z = t / (2.0 + t)
    z2 = z * z
    log1p_t = 2.0 * z * (1.0 + z2 * (1.0 / 3.0 + z2 * (
        1.0 / 5.0 + z2 * (1.0 / 7.0 + z2 * (1.0 / 9.0)))))
    sp = jnp.maximum(a, 0.0) + log1p_t
    e = jnp.exp(-1.702 * sp * (th - b))
    return cs + (1.0 - cs) / (1.0 + e)


def _make_sc_kernel(bpw, n_cores):
    mesh = plsc.VectorSubcoreMesh(core_axis_name="c", subcore_axis_name="s")

    @functools.partial(
        pl.kernel,
        mesh=mesh,
        out_type=jax.ShapeDtypeStruct((BATCH,), jnp.float32),
        scratch_types=[
            pltpu.VMEM((bpw,), jnp.int32),
            pltpu.VMEM((bpw,), jnp.int32),
            pltpu.VMEM((bpw,), jnp.float32),
            pltpu.VMEM((bpw,), jnp.float32),
            pltpu.VMEM((bpw,), jnp.float32),
            pltpu.VMEM((bpw,), jnp.float32),
            pltpu.VMEM((bpw,), jnp.float32),
            pltpu.SemaphoreType.DMA,
            pltpu.SemaphoreType.DMA,
            pltpu.SemaphoreType.DMA,
            pltpu.SemaphoreType.DMA,
            pltpu.SemaphoreType.DMA,
            pltpu.SemaphoreType.DMA,
        ],
    )
    def sc_kernel(sid_hbm, eid_hbm, th_hbm, a_hbm, b_hbm, c_hbm, out_hbm,
                  sidx_v, eidx_v, th_v, a_v, b_v, c_v, out_v,
                  sem0, sem1, sem2, sem3, sem4, sem5):
        wid = lax.axis_index("s") * n_cores + lax.axis_index("c")
        base = wid * bpw
        th_t = th_hbm.at[0]
        a_t = a_hbm.at[0]
        b_t = b_hbm.at[0]
        c_t = c_hbm.at[0]
        i0 = pltpu.async_copy(sid_hbm.at[pl.ds(base, bpw)], sidx_v, sem4)
        i1 = pltpu.async_copy(eid_hbm.at[pl.ds(base, bpw)], eidx_v, sem5)
        i1.wait()
        g1 = pltpu.async_copy(a_t.at[eidx_v], a_v, sem1)
        g2 = pltpu.async_copy(b_t.at[eidx_v], b_v, sem2)
        g3 = pltpu.async_copy(c_t.at[eidx_v], c_v, sem3)
        i0.wait()
        g0 = pltpu.async_copy(th_t.at[sidx_v], th_v, sem0)
        g1.wait()
        g2.wait()
        g3.wait()
        g0.wait()

        def step(i, _):
            sl = pl.ds(i * LANES, LANES)
            out_v[sl] = _irt_formula(th_v[sl], a_v[sl], b_v[sl], c_v[sl])
            return _

        lax.fori_loop(0, bpw // LANES, step, None, unroll=True)
        pltpu.sync_copy(out_v, out_hbm.at[pl.ds(base, bpw)])

    return sc_kernel


def kernel(student_id, exercise_id, theta_table, a_table, b_table, c_table):
    info = plsc.get_sparse_core_info()
    n_workers = info.num_cores * info.num_subcores
    bpw = BATCH // n_workers
    sc_kernel = _make_sc_kernel(bpw, info.num_cores)
    return sc_kernel(
        student_id.astype(jnp.int32),
        exercise_id.astype(jnp.int32),
        theta_table.T,
        a_table.T,
        b_table.T,
        c_table.T,
    )


# speedup vs baseline: 1.0640x; 1.0640x over previous
import functools

import jax
import jax.numpy as jnp
from jax import lax
from jax.experimental import pallas as pl
from jax.experimental.pallas import tpu as pltpu
from jax.experimental.pallas import tpu_sc as plsc

BATCH = 16384
LANES = 16


def _irt_formula(th, a, b, c):
    cs = 1.0 / (1.0 + jnp.exp(-c))
    t = jnp.exp(-jnp.abs(a))
    z = t / (2.0 + t)
    z2 = z * z
    log1p_t = 2.0 * z * (1.0 + z2 * (1.0 / 3.0 + z2 * (
        1.0 / 5.0 + z2 * (1.0 / 7.0 + z2 * (1.0 / 9.0)))))
    sp = jnp.maximum(a, 0.0) + log1p_t
    e = jnp.exp(-1.702 * sp * (th - b))
    return cs + (1.0 - cs) / (1.0 + e)


def _make_sc_kernel(bpw, n_cores):
    mesh = plsc.VectorSubcoreMesh(core_axis_name="c", subcore_axis_name="s")

    @functools.partial(
        pl.kernel,
        mesh=mesh,
        out_type=jax.ShapeDtypeStruct((BATCH,), jnp.float32),
        scratch_types=[
            pltpu.VMEM((bpw,), jnp.int32),
            pltpu.VMEM((bpw,), jnp.int32),
            pltpu.VMEM((bpw,), jnp.float32),
            pltpu.VMEM((bpw,), jnp.float32),
            pltpu.VMEM((bpw,), jnp.float32),
            pltpu.VMEM((bpw,), jnp.float32),
            pltpu.VMEM((bpw,), jnp.float32),
            pltpu.SemaphoreType.DMA,
            pltpu.SemaphoreType.DMA,
            pltpu.SemaphoreType.DMA,
            pltpu.SemaphoreType.DMA,
            pltpu.SemaphoreType.DMA,
            pltpu.SemaphoreType.DMA,
        ],
    )
    def sc_kernel(sid_hbm, eid_hbm, th_hbm, a_hbm, b_hbm, c_hbm, out_hbm,
                  sidx_v, eidx_v, th_v, a_v, b_v, c_v, out_v,
                  sem0, sem1, sem2, sem3, sem4, sem5):
        wid = lax.axis_index("s") * n_cores + lax.axis_index("c")
        base = wid * bpw
        th_t = th_hbm.at[0]
        a_t = a_hbm.at[0]
        b_t = b_hbm.at[0]
        c_t = c_hbm.at[0]
        i0 = pltpu.async_copy(sid_hbm.at[pl.ds(base, bpw)], sidx_v, sem4)
        i1 = pltpu.async_copy(eid_hbm.at[pl.ds(base, bpw)], eidx_v, sem5)
        i1.wait()
        g1 = pltpu.async_copy(a_t.at[eidx_v], a_v, sem1)
        g2 = pltpu.async_copy(b_t.at[eidx_v], b_v, sem2)
        g3 = pltpu.async_copy(c_t.at[eidx_v], c_v, sem3)
        i0.wait()
        half = bpw // 2
        g0a = pltpu.async_copy(th_t.at[sidx_v.at[pl.ds(0, half)]],
                               th_v.at[pl.ds(0, half)], sem0)
        g0b = pltpu.async_copy(th_t.at[sidx_v.at[pl.ds(half, half)]],
                               th_v.at[pl.ds(half, half)], sem5)
        g1.wait()
        g2.wait()
        g3.wait()
        g0a.wait()

        def step(i, _):
            sl = pl.ds(i * LANES, LANES)
            out_v[sl] = _irt_formula(th_v[sl], a_v[sl], b_v[sl], c_v[sl])
            return _

        nh = half // LANES
        lax.fori_loop(0, nh, step, None, unroll=4)
        g0b.wait()
        lax.fori_loop(nh, 2 * nh, step, None, unroll=4)
        pltpu.sync_copy(out_v, out_hbm.at[pl.ds(base, bpw)])

    return sc_kernel


def kernel(student_id, exercise_id, theta_table, a_table, b_table, c_table):
    info = plsc.get_sparse_core_info()
    n_workers = info.num_cores * info.num_subcores
    bpw = BATCH // n_workers
    sc_kernel = _make_sc_kernel(bpw, info.num_cores)
    return sc_kernel(
        student_id.astype(jnp.int32),
        exercise_id.astype(jnp.int32),
        theta_table.T,
        a_table.T,
        b_table.T,
        c_table.T,
    )


# speedup vs baseline: 1.0791x; 1.0141x over previous
import functools

import jax
import jax.numpy as jnp
from jax import lax
from jax.experimental import pallas as pl
from jax.experimental.pallas import tpu as pltpu
from jax.experimental.pallas import tpu_sc as plsc

BATCH = 16384
LANES = 16


def _irt_formula(th, a, b, c):
    cs = 1.0 / (1.0 + jnp.exp(-c))
    t = jnp.exp(-jnp.abs(a))
    z = t / (2.0 + t)
    z2 = z * z
    log1p_t = 2.0 * z * (1.0 + z2 * (1.0 / 3.0 + z2 * (
        1.0 / 5.0 + z2 * (1.0 / 7.0 + z2 * (1.0 / 9.0)))))
    sp = jnp.maximum(a, 0.0) + log1p_t
    e = jnp.exp(-1.702 * sp * (th - b))
    return cs + (1.0 - cs) / (1.0 + e)


def _make_sc_kernel(bpw, n_cores):
    mesh = plsc.VectorSubcoreMesh(core_axis_name="c", subcore_axis_name="s")

    @functools.partial(
        pl.kernel,
        mesh=mesh,
        out_type=jax.ShapeDtypeStruct((BATCH,), jnp.float32),
        scratch_types=[
            pltpu.VMEM((bpw,), jnp.int32),
            pltpu.VMEM((bpw,), jnp.int32),
            pltpu.VMEM((bpw,), jnp.float32),
            pltpu.VMEM((bpw,), jnp.float32),
            pltpu.VMEM((bpw,), jnp.float32),
            pltpu.VMEM((bpw,), jnp.float32),
            pltpu.VMEM((bpw,), jnp.float32),
            pltpu.SemaphoreType.DMA,
            pltpu.SemaphoreType.DMA,
            pltpu.SemaphoreType.DMA,
            pltpu.SemaphoreType.DMA,
            pltpu.SemaphoreType.DMA,
            pltpu.SemaphoreType.DMA,
        ],
    )
    def sc_kernel(sid_hbm, eid_hbm, th_hbm, a_hbm, b_hbm, c_hbm, out_hbm,
                  sidx_v, eidx_v, th_v, a_v, b_v, c_v, out_v,
                  sem0, sem1, sem2, sem3, sem4, sem5):
        wid = lax.axis_index("s") * n_cores + lax.axis_index("c")
        base = wid * bpw
        th_t = th_hbm.at[0]
        a_t = a_hbm.at[0]
        b_t = b_hbm.at[0]
        c_t = c_hbm.at[0]
        i0 = pltpu.async_copy(sid_hbm.at[pl.ds(base, bpw)], sidx_v, sem4)
        i1 = pltpu.async_copy(eid_hbm.at[pl.ds(base, bpw)], eidx_v, sem5)
        i1.wait()
        g1 = pltpu.async_copy(a_t.at[eidx_v], a_v, sem1)
        g2 = pltpu.async_copy(b_t.at[eidx_v], b_v, sem2)
        g3 = pltpu.async_copy(c_t.at[eidx_v], c_v, sem3)
        i0.wait()
        g0 = pltpu.async_copy(th_t.at[sidx_v], th_v, sem0)
        g1.wait()
        g2.wait()
        g3.wait()
        g0.wait()

        def step(i, _):
            sl = pl.ds(i * LANES, LANES)
            out_v[sl] = _irt_formula(th_v[sl], a_v[sl], b_v[sl], c_v[sl])
            return _

        lax.fori_loop(0, bpw // LANES, step, None, unroll=4)
        pltpu.sync_copy(out_v, out_hbm.at[pl.ds(base, bpw)])

    return sc_kernel


def kernel(student_id, exercise_id, theta_table, a_table, b_table, c_table):
    info = plsc.get_sparse_core_info()
    n_workers = info.num_cores * info.num_subcores
    bpw = BATCH // n_workers
    sc_kernel = _make_sc_kernel(bpw, info.num_cores)
    return sc_kernel(
        student_id.astype(jnp.int32),
        exercise_id.astype(jnp.int32),
        theta_table.T,
        a_table.T,
        b_table.T,
        c_table.T,
    )
